# baseline (speedup 1.0000x reference)
import functools
import math

import jax
import jax.numpy as jnp
from jax import lax
from jax.experimental import pallas as pl
from jax.experimental.pallas import tpu as pltpu
from jax.experimental.pallas import tpu_sc as plsc


def _sc_geometry():
    try:
        info = plsc.get_sparse_core_info()
        return info.num_cores, info.num_subcores
    except Exception:
        return 2, 16


@functools.lru_cache(maxsize=None)
def _build(B, V, D):
    NC, NS = _sc_geometry()
    NW = NC * NS
    assert B % NW == 0
    b_per_w = B // NW
    CHUNK = 128
    assert b_per_w % CHUNK == 0
    n_chunks = b_per_w // CHUNK
    scale = math.sqrt(D)
    assert D % 16 == 0
    d_vecs = D // 16

    mesh = plsc.VectorSubcoreMesh(core_axis_name="c", subcore_axis_name="s")

    @functools.partial(
        pl.kernel,
        out_type=jax.ShapeDtypeStruct((B, D), jnp.float32),
        mesh=mesh,
        scratch_types=[
            pltpu.VMEM((CHUNK,), jnp.int32),
            pltpu.VMEM((CHUNK, D), jnp.float32),
            pltpu.SemaphoreType.DMA,
        ],
        compiler_params=pltpu.CompilerParams(use_tc_tiling_on_sc=False),
    )
    def emb_kernel(tokens_hbm, table_hbm, out_hbm, idx_v, rows_v, sem):
        wid = lax.axis_index("s") * NC + lax.axis_index("c")
        base = wid * b_per_w

        def chunk_body(g, _):
            off = base + g * CHUNK
            pltpu.sync_copy(tokens_hbm.at[pl.ds(off, CHUNK)], idx_v)
            pltpu.async_copy(table_hbm.at[idx_v], rows_v, sem).wait()

            def scale_body(i, _):
                for j in range(d_vecs):
                    sl = pl.ds(j * 16, 16)
                    rows_v[i, sl] = rows_v[i, sl] * scale
                return ()

            lax.fori_loop(0, CHUNK, scale_body, ())
            pltpu.sync_copy(rows_v, out_hbm.at[pl.ds(off, CHUNK)])
            return ()

        lax.fori_loop(0, n_chunks, chunk_body, ())

    return emb_kernel


def kernel(tokens, table):
    batch, seq = tokens.shape
    V, D = table.shape
    B = batch * seq
    tok_flat = tokens.reshape(B).astype(jnp.int32)
    out = _build(B, V, D)(tok_flat, table)
    return out.reshape(batch, seq, D)


# speedup vs baseline: 1.0076x; 1.0076x over previous
import functools
import math

import jax
import jax.numpy as jnp
from jax import lax
from jax.experimental import pallas as pl
from jax.experimental.pallas import tpu as pltpu
from jax.experimental.pallas import tpu_sc as plsc


def _sc_geometry():
    try:
        info = plsc.get_sparse_core_info()
        return info.num_cores, info.num_subcores
    except Exception:
        return 2, 16


@functools.lru_cache(maxsize=None)
def _build(B, V, D):
    NC, NS = _sc_geometry()
    NW = NC * NS
    assert B % NW == 0
    b_per_w = B // NW
    CHUNK = 128
    NBUF = 4
    assert b_per_w % (CHUNK * NBUF) == 0
    n_chunks = b_per_w // CHUNK
    n_steps = n_chunks // NBUF
    scale = math.sqrt(D)
    assert D % 16 == 0
    d_vecs = D // 16

    mesh = plsc.VectorSubcoreMesh(core_axis_name="c", subcore_axis_name="s")

    @functools.partial(
        pl.kernel,
        out_type=jax.ShapeDtypeStruct((B, D), jnp.float32),
        mesh=mesh,
        scratch_types=[
            pltpu.VMEM((n_chunks, CHUNK), jnp.int32),
            [pltpu.VMEM((CHUNK, D), jnp.float32) for _ in range(NBUF)],
            [pltpu.VMEM((CHUNK, D), jnp.float32) for _ in range(NBUF)],
            [pltpu.SemaphoreType.DMA for _ in range(NBUF)],
            [pltpu.SemaphoreType.DMA for _ in range(NBUF)],
        ],
        compiler_params=pltpu.CompilerParams(use_tc_tiling_on_sc=False),
    )
    def emb_kernel(tokens_hbm, table_hbm, out_hbm, idx_v, rows_g, rows_s,
                   sem_g, sem_s):
        wid = lax.axis_index("s") * NC + lax.axis_index("c")
        base = wid * b_per_w

        pltpu.sync_copy(tokens_hbm.at[wid], idx_v)

        for b in range(NBUF):
            pltpu.async_copy(table_hbm.at[idx_v.at[b]], rows_g[b], sem_g[b])

        def step_body(step, _):
            for b in range(NBUF):
                g = step * NBUF + b
                off = base + g * CHUNK
                pltpu.make_async_copy(
                    table_hbm.at[idx_v.at[g]], rows_g[b], sem_g[b]).wait()

                @pl.when(step > 0)
                def _wait_prev_scatter(b=b):
                    pltpu.make_async_copy(
                        rows_s[b], out_hbm.at[pl.ds(base, CHUNK)],
                        sem_s[b]).wait()

                src, dst = rows_g[b], rows_s[b]

                @pl.loop(0, CHUNK, unroll=8)
                def _scale(i):
                    for j in range(d_vecs):
                        sl = pl.ds(j * 16, 16)
                        dst[i, sl] = src[i, sl] * scale

                pltpu.async_copy(dst, out_hbm.at[pl.ds(off, CHUNK)], sem_s[b])

                g2 = g + NBUF

                @pl.when(g2 < n_chunks)
                def _next_gather(b=b, g2=g2):
                    pltpu.async_copy(
                        table_hbm.at[idx_v.at[g2]], rows_g[b], sem_g[b])
            return ()

        lax.fori_loop(0, n_steps, step_body, ())

        for b in range(NBUF):
            pltpu.make_async_copy(
                rows_s[b], out_hbm.at[pl.ds(base, CHUNK)], sem_s[b]).wait()

    return emb_kernel


def kernel(tokens, table):
    batch, seq = tokens.shape
    V, D = table.shape
    B = batch * seq
    NC, NS = _sc_geometry()
    NW = NC * NS
    b_per_w = B // NW
    CHUNK = 128
    tok = tokens.reshape(NW, b_per_w // CHUNK, CHUNK).astype(jnp.int32)
    out = _build(B, V, D)(tok, table)
    return out.reshape(batch, seq, D)
